# baseline (speedup 1.0000x reference)
import functools

import jax
import jax.numpy as jnp
from jax import lax
from jax.experimental import pallas as pl
from jax.experimental.pallas import tpu as pltpu
from jax.experimental.pallas import tpu_sc as plsc

D = 1024
E = 8
K = 2
BAL = 1e-4
N_TOK = 4096
BLK = 256
TILE = 256
T_MAX = N_TOK * K // TILE + E
P_MAX = T_MAX * TILE
NW = 32
CH = 64


def _router_body(x_ref, wr_ref, br_ref, ti_ref, tw_ref, rk_ref, cnt_ref,
                 loss_ref, acc_ref, cacc_ref):
    i = pl.program_id(0)
    nb = pl.num_programs(0)

    @pl.when(i == 0)
    def _():
        acc_ref[...] = jnp.zeros_like(acc_ref)
        cacc_ref[...] = jnp.zeros_like(cacc_ref)

    x = x_ref[...]
    logits = jnp.dot(x.astype(jnp.bfloat16), wr_ref[...].astype(jnp.bfloat16),
                     preferred_element_type=jnp.float32) + br_ref[...]
    m = jnp.max(logits, axis=1, keepdims=True)
    ex = jnp.exp(logits - m)
    probs = ex / jnp.sum(ex, axis=1, keepdims=True)
    iota = lax.broadcasted_iota(jnp.int32, (BLK, E), 1)
    m1 = jnp.max(probs, axis=1, keepdims=True)
    i1 = jnp.min(jnp.where(probs == m1, iota, E), axis=1, keepdims=True)
    masked = jnp.where(iota == i1, -1.0, probs)
    m2 = jnp.max(masked, axis=1, keepdims=True)
    i2 = jnp.min(jnp.where(masked == m2, iota, E), axis=1, keepdims=True)
    ti_ref[:, 0:1] = i1
    ti_ref[:, 1:2] = i2
    tw_ref[:, 0:1] = m1
    tw_ref[:, 1:2] = m2
    oh0 = (iota == i1).astype(jnp.float32)
    oh1 = (iota == i2).astype(jnp.float32)

    r = lax.broadcasted_iota(jnp.int32, (BLK, BLK), 0)
    c = lax.broadcasted_iota(jnp.int32, (BLK, BLK), 1)
    tril = (c < r).astype(jnp.bfloat16)
    s01 = jnp.dot(tril, (oh0 + oh1).astype(jnp.bfloat16),
                  preferred_element_type=jnp.float32)
    base0 = cacc_ref[...] + s01
    base1 = base0 + oh0
    rk_ref[:, 0:1] = jnp.sum(oh0 * base0, axis=1, keepdims=True).astype(jnp.int32)
    rk_ref[:, 1:2] = jnp.sum(oh1 * base1, axis=1, keepdims=True).astype(jnp.int32)

    acc_ref[...] += jnp.sum(probs, axis=0, keepdims=True)
    cacc_ref[...] += jnp.sum(oh0 + oh1, axis=0, keepdims=True)

    @pl.when(i == nb - 1)
    def _():
        s = acc_ref[...] / N_TOK
        loss_ref[...] = jnp.sum((1.0 / E - s) ** 2, axis=1, keepdims=True) \
            * (BAL / E)
        cnt_ref[...] = cacc_ref[...]


def _ffn_body(te_ref, act_ref, xs_ref, w1_ref, b1_ref, w2_ref, b2_ref, ys_ref,
              w1b_ref, w2b_ref):
    i = pl.program_id(0)
    changed = jnp.logical_or(i == 0,
                             te_ref[i] != te_ref[jnp.maximum(i - 1, 0)])

    @pl.when(jnp.logical_and(act_ref[i] == 1, changed))
    def _():
        w1b_ref[...] = w1_ref[0].astype(jnp.bfloat16)
        w2b_ref[...] = w2_ref[0].astype(jnp.bfloat16)

    @pl.when(act_ref[i] == 1)
    def _():
        xb = xs_ref[...].astype(jnp.bfloat16)
        h = jnp.dot(xb, w1b_ref[...],
                    preferred_element_type=jnp.float32) + b1_ref[0]
        hb = h.astype(jnp.bfloat16)
        hb = jnp.where(hb >= 0, hb, jnp.bfloat16(0.01) * hb)
        y = jnp.dot(hb, w2b_ref[...],
                    preferred_element_type=jnp.float32) + b2_ref[0]
        ys_ref[...] = jnp.where(y >= 0, y, 0.01 * y)


def _comb_body(g0_ref, g1_ref, w_ref, o_ref):
    w = w_ref[...]
    o_ref[...] = g0_ref[...] * w[:, 0:1] + g1_ref[...] * w[:, 1:2]


def _slot_add(off_v, e_v, r_v, p_v):
    @pl.loop(0, CH, step=16)
    def _(k):
        ev = e_v[pl.ds(k, 16)]
        pv = r_v[pl.ds(k, 16)] + plsc.load_gather(off_v, [ev])
        p_v[pl.ds(k, 16)] = pv


@functools.cache
def _sc_kernels():
    mesh = plsc.VectorSubcoreMesh(core_axis_name="c", subcore_axis_name="s")
    cp = pltpu.CompilerParams(needs_layout_passes=False)

    @functools.partial(
        pl.kernel, mesh=mesh, compiler_params=cp,
        out_type=jax.ShapeDtypeStruct((P_MAX, D), jnp.float32),
        scratch_types=[pltpu.VMEM((CH, D), jnp.float32),
                       pltpu.VMEM((CH,), jnp.int32),
                       pltpu.VMEM((CH,), jnp.int32),
                       pltpu.VMEM((CH,), jnp.int32),
                       pltpu.VMEM((16,), jnp.int32)])
    def sc_scatter(x_hbm, e0_hbm, e1_hbm, r0_hbm, r1_hbm, off_hbm, xs_hbm,
                   rows_v, e_v, r_v, p_v, off_s):
        wid = lax.axis_index("s") * 2 + lax.axis_index("c")
        base = wid * (N_TOK // NW)
        pltpu.sync_copy(off_hbm, off_s)

        @pl.loop(0, N_TOK // NW, step=CH)
        def _(j):
            pltpu.sync_copy(x_hbm.at[pl.ds(base + j, CH)], rows_v)
            pltpu.sync_copy(e0_hbm.at[pl.ds(base + j, CH)], e_v)
            pltpu.sync_copy(r0_hbm.at[pl.ds(base + j, CH)], r_v)
            _slot_add(off_s, e_v, r_v, p_v)
            pltpu.sync_copy(rows_v, xs_hbm.at[p_v])
            pltpu.sync_copy(e1_hbm.at[pl.ds(base + j, CH)], e_v)
            pltpu.sync_copy(r1_hbm.at[pl.ds(base + j, CH)], r_v)
            _slot_add(off_s, e_v, r_v, p_v)
            pltpu.sync_copy(rows_v, xs_hbm.at[p_v])

    @functools.partial(
        pl.kernel, mesh=mesh, compiler_params=cp,
        out_type=jax.ShapeDtypeStruct((K * N_TOK, D), jnp.float32),
        scratch_types=[pltpu.VMEM((CH, D), jnp.float32),
                       pltpu.VMEM((CH,), jnp.int32),
                       pltpu.VMEM((CH,), jnp.int32),
                       pltpu.VMEM((CH,), jnp.int32),
                       pltpu.VMEM((16,), jnp.int32),
                       pltpu.SemaphoreType.DMA])
    def sc_gather(ys_hbm, ea_hbm, ra_hbm, off_hbm, g_hbm,
                  rows_v, e_v, r_v, p_v, off_s, sem):
        wid = lax.axis_index("s") * 2 + lax.axis_index("c")
        base = wid * (K * N_TOK // NW)
        pltpu.sync_copy(off_hbm, off_s)

        @pl.loop(0, K * N_TOK // NW, step=CH)
        def _(j):
            pltpu.sync_copy(ea_hbm.at[pl.ds(base + j, CH)], e_v)
            pltpu.sync_copy(ra_hbm.at[pl.ds(base + j, CH)], r_v)
            _slot_add(off_s, e_v, r_v, p_v)
            pltpu.async_copy(ys_hbm.at[p_v], rows_v, sem).wait()
            pltpu.sync_copy(rows_v, g_hbm.at[pl.ds(base + j, CH)])

    return sc_scatter, sc_gather


def kernel(x, Wr, br, W1, b1, W2, b2):
    B, T, _ = x.shape
    x_flat = x.reshape(B * T, D)

    ti, tw, rk, cnt, loss = pl.pallas_call(
        _router_body,
        grid=(N_TOK // BLK,),
        in_specs=[
            pl.BlockSpec((BLK, D), lambda i: (i, 0)),
            pl.BlockSpec((D, E), lambda i: (0, 0)),
            pl.BlockSpec((1, E), lambda i: (0, 0)),
        ],
        out_specs=[
            pl.BlockSpec((BLK, K), lambda i: (i, 0)),
            pl.BlockSpec((BLK, K), lambda i: (i, 0)),
            pl.BlockSpec((BLK, K), lambda i: (i, 0)),
            pl.BlockSpec((1, E), lambda i: (0, 0)),
            pl.BlockSpec((1, 1), lambda i: (0, 0)),
        ],
        out_shape=[
            jax.ShapeDtypeStruct((N_TOK, K), jnp.int32),
            jax.ShapeDtypeStruct((N_TOK, K), jnp.float32),
            jax.ShapeDtypeStruct((N_TOK, K), jnp.int32),
            jax.ShapeDtypeStruct((1, E), jnp.float32),
            jax.ShapeDtypeStruct((1, 1), jnp.float32),
        ],
        scratch_shapes=[pltpu.VMEM((1, E), jnp.float32),
                        pltpu.VMEM((1, E), jnp.float32)],
    )(x_flat, Wr, br.reshape(1, E))

    c = cnt[0].astype(jnp.int32)
    padded = ((c + TILE - 1) // TILE) * TILE
    ends = jnp.cumsum(padded)
    off16 = jnp.zeros((16,), jnp.int32).at[0:E].set(ends - padded)

    t_start = jnp.arange(T_MAX, dtype=jnp.int32) * TILE
    tile_e_raw = jnp.sum((t_start[:, None] >= ends[None, :]).astype(jnp.int32),
                         axis=1)
    last_e = jnp.max(jnp.where(padded > 0, jnp.arange(E, dtype=jnp.int32), -1))
    tile_e = jnp.minimum(tile_e_raw, last_e)
    active = (t_start < ends[E - 1]).astype(jnp.int32)

    sc_scatter, sc_gather = _sc_kernels()
    xs = sc_scatter(x_flat, ti[:, 0], ti[:, 1], rk[:, 0], rk[:, 1], off16)

    ys = pl.pallas_call(
        _ffn_body,
        grid_spec=pltpu.PrefetchScalarGridSpec(
            num_scalar_prefetch=2,
            grid=(T_MAX,),
            in_specs=[
                pl.BlockSpec((TILE, D), lambda i, te, act: (i, 0)),
                pl.BlockSpec((1, D, 2 * D), lambda i, te, act: (te[i], 0, 0)),
                pl.BlockSpec((1, 1, 2 * D), lambda i, te, act: (te[i], 0, 0)),
                pl.BlockSpec((1, 2 * D, D), lambda i, te, act: (te[i], 0, 0)),
                pl.BlockSpec((1, 1, D), lambda i, te, act: (te[i], 0, 0)),
            ],
            out_specs=pl.BlockSpec((TILE, D), lambda i, te, act: (i, 0)),
            scratch_shapes=[pltpu.VMEM((D, 2 * D), jnp.bfloat16),
                            pltpu.VMEM((2 * D, D), jnp.bfloat16)],
        ),
        out_shape=jax.ShapeDtypeStruct((P_MAX, D), jnp.float32),
    )(tile_e, active, xs, W1, b1.reshape(E, 1, 2 * D), W2,
      b2.reshape(E, 1, D))

    ea = jnp.concatenate([ti[:, 0], ti[:, 1]])
    ra = jnp.concatenate([rk[:, 0], rk[:, 1]])
    g = sc_gather(ys, ea, ra, off16)

    out_flat = pl.pallas_call(
        _comb_body,
        grid=(N_TOK // BLK,),
        in_specs=[
            pl.BlockSpec((BLK, D), lambda i: (i, 0)),
            pl.BlockSpec((BLK, D), lambda i: (i + N_TOK // BLK, 0)),
            pl.BlockSpec((BLK, K), lambda i: (i, 0)),
        ],
        out_specs=pl.BlockSpec((BLK, D), lambda i: (i, 0)),
        out_shape=jax.ShapeDtypeStruct((N_TOK, D), jnp.float32),
    )(g, g, tw)

    return out_flat.reshape(B, T, D), loss.reshape(())


# speedup vs baseline: 1.0681x; 1.0681x over previous
import functools

import jax
import jax.numpy as jnp
from jax import lax
from jax.experimental import pallas as pl
from jax.experimental.pallas import tpu as pltpu
from jax.experimental.pallas import tpu_sc as plsc

D = 1024
E = 8
K = 2
BAL = 1e-4
N_TOK = 4096
BLK = 256
TILE = 256
T_MAX = N_TOK * K // TILE + E
TPE = N_TOK // TILE
P_ALL = E * N_TOK
NW = 32
CH = 64


def _router_body(x_ref, wr_ref, br_ref, pos_ref, tw_ref, cnt_ref,
                 loss_ref, acc_ref, cacc_ref):
    i = pl.program_id(0)
    nb = pl.num_programs(0)

    @pl.when(i == 0)
    def _():
        acc_ref[...] = jnp.zeros_like(acc_ref)
        cacc_ref[...] = jnp.zeros_like(cacc_ref)

    x = x_ref[...]
    logits = jnp.dot(x.astype(jnp.bfloat16), wr_ref[...].astype(jnp.bfloat16),
                     preferred_element_type=jnp.float32) + br_ref[...]
    m = jnp.max(logits, axis=1, keepdims=True)
    ex = jnp.exp(logits - m)
    probs = ex / jnp.sum(ex, axis=1, keepdims=True)
    iota = lax.broadcasted_iota(jnp.int32, (BLK, E), 1)
    m1 = jnp.max(probs, axis=1, keepdims=True)
    i1 = jnp.min(jnp.where(probs == m1, iota, E), axis=1, keepdims=True)
    masked = jnp.where(iota == i1, -1.0, probs)
    m2 = jnp.max(masked, axis=1, keepdims=True)
    i2 = jnp.min(jnp.where(masked == m2, iota, E), axis=1, keepdims=True)
    tw_ref[:, 0:1] = m1
    tw_ref[:, 1:2] = m2
    oh0 = (iota == i1).astype(jnp.float32)
    oh1 = (iota == i2).astype(jnp.float32)

    r = lax.broadcasted_iota(jnp.int32, (BLK, BLK), 0)
    c = lax.broadcasted_iota(jnp.int32, (BLK, BLK), 1)
    tril = (c < r).astype(jnp.bfloat16)
    s01 = jnp.dot(tril, (oh0 + oh1).astype(jnp.bfloat16),
                  preferred_element_type=jnp.float32)
    base0 = cacc_ref[...] + s01
    base1 = base0 + oh0
    rk0 = jnp.sum(oh0 * base0, axis=1, keepdims=True).astype(jnp.int32)
    rk1 = jnp.sum(oh1 * base1, axis=1, keepdims=True).astype(jnp.int32)
    pos_ref[:, 0:1] = rk0 + i1 * N_TOK
    pos_ref[:, 1:2] = rk1 + i2 * N_TOK

    acc_ref[...] += jnp.sum(probs, axis=0, keepdims=True)
    cacc_ref[...] += jnp.sum(oh0 + oh1, axis=0, keepdims=True)

    @pl.when(i == nb - 1)
    def _():
        s = acc_ref[...] / N_TOK
        loss_ref[...] = jnp.sum((1.0 / E - s) ** 2, axis=1, keepdims=True) \
            * (BAL / E)
        cnt_ref[...] = cacc_ref[...]


def _ffn_body(te_ref, act_ref, xbi_ref, xs_ref, w1_ref, b1_ref, w2_ref,
              b2_ref, ys_ref, w1b_ref, w2b_ref):
    i = pl.program_id(0)
    changed = jnp.logical_or(i == 0,
                             te_ref[i] != te_ref[jnp.maximum(i - 1, 0)])

    @pl.when(jnp.logical_and(act_ref[i] == 1, changed))
    def _():
        w1b_ref[...] = w1_ref[0].astype(jnp.bfloat16)
        w2b_ref[...] = w2_ref[0].astype(jnp.bfloat16)

    @pl.when(act_ref[i] == 1)
    def _():
        xb = xs_ref[...].astype(jnp.bfloat16)
        h = jnp.dot(xb, w1b_ref[...],
                    preferred_element_type=jnp.float32) + b1_ref[0]
        hb = h.astype(jnp.bfloat16)
        hb = jnp.where(hb >= 0, hb, jnp.bfloat16(0.01) * hb)
        y = jnp.dot(hb, w2b_ref[...],
                    preferred_element_type=jnp.float32) + b2_ref[0]
        ys_ref[...] = jnp.where(y >= 0, y, 0.01 * y)


def _comb_body(g0_ref, g1_ref, w_ref, o_ref):
    w = w_ref[...]
    o_ref[...] = g0_ref[...] * w[:, 0:1] + g1_ref[...] * w[:, 1:2]


@functools.cache
def _sc_kernels():
    mesh = plsc.VectorSubcoreMesh(core_axis_name="c", subcore_axis_name="s")

    @functools.partial(
        pl.kernel, mesh=mesh,
        out_type=jax.ShapeDtypeStruct((P_ALL, D), jnp.float32),
        scratch_types=[pltpu.VMEM((CH, D), jnp.float32),
                       pltpu.VMEM((CH,), jnp.int32),
                       pltpu.VMEM((CH,), jnp.int32)])
    def sc_scatter(x_hbm, p0_hbm, p1_hbm, xs_hbm, rows_v, i0_v, i1_v):
        wid = lax.axis_index("s") * 2 + lax.axis_index("c")
        base = wid * (N_TOK // NW)

        @pl.loop(0, N_TOK // NW, step=CH)
        def _(j):
            pltpu.sync_copy(x_hbm.at[pl.ds(base + j, CH)], rows_v)
            pltpu.sync_copy(p0_hbm.at[pl.ds(base + j, CH)], i0_v)
            pltpu.sync_copy(p1_hbm.at[pl.ds(base + j, CH)], i1_v)
            pltpu.sync_copy(rows_v, xs_hbm.at[i0_v])
            pltpu.sync_copy(rows_v, xs_hbm.at[i1_v])

    @functools.partial(
        pl.kernel, mesh=mesh,
        out_type=jax.ShapeDtypeStruct((K * N_TOK, D), jnp.float32),
        scratch_types=[pltpu.VMEM((CH, D), jnp.float32),
                       pltpu.VMEM((CH,), jnp.int32),
                       pltpu.SemaphoreType.DMA])
    def sc_gather(ys_hbm, idx_hbm, g_hbm, rows_v, i_v, sem):
        wid = lax.axis_index("s") * 2 + lax.axis_index("c")
        base = wid * (K * N_TOK // NW)

        @pl.loop(0, K * N_TOK // NW, step=CH)
        def _(j):
            pltpu.sync_copy(idx_hbm.at[pl.ds(base + j, CH)], i_v)
            pltpu.async_copy(ys_hbm.at[i_v], rows_v, sem).wait()
            pltpu.sync_copy(rows_v, g_hbm.at[pl.ds(base + j, CH)])

    return sc_scatter, sc_gather


def kernel(x, Wr, br, W1, b1, W2, b2):
    B, T, _ = x.shape
    x_flat = x.reshape(B * T, D)

    pos, tw, cnt, loss = pl.pallas_call(
        _router_body,
        grid=(N_TOK // BLK,),
        in_specs=[
            pl.BlockSpec((BLK, D), lambda i: (i, 0)),
            pl.BlockSpec((D, E), lambda i: (0, 0)),
            pl.BlockSpec((1, E), lambda i: (0, 0)),
        ],
        out_specs=[
            pl.BlockSpec((BLK, K), lambda i: (i, 0)),
            pl.BlockSpec((BLK, K), lambda i: (i, 0)),
            pl.BlockSpec((1, E), lambda i: (0, 0)),
            pl.BlockSpec((1, 1), lambda i: (0, 0)),
        ],
        out_shape=[
            jax.ShapeDtypeStruct((N_TOK, K), jnp.int32),
            jax.ShapeDtypeStruct((N_TOK, K), jnp.float32),
            jax.ShapeDtypeStruct((1, E), jnp.float32),
            jax.ShapeDtypeStruct((1, 1), jnp.float32),
        ],
        scratch_shapes=[pltpu.VMEM((1, E), jnp.float32),
                        pltpu.VMEM((1, E), jnp.float32)],
    )(x_flat, Wr, br.reshape(1, E))

    sc_scatter, sc_gather = _sc_kernels()
    xs = sc_scatter(x_flat, pos[:, 0], pos[:, 1])

    c = cnt[0].astype(jnp.int32)
    nt = (c + TILE - 1) // TILE
    cum = jnp.cumsum(nt)
    cumprev = cum - nt
    t_idx = jnp.arange(T_MAX, dtype=jnp.int32)
    te_raw = jnp.sum((t_idx[:, None] >= cum[None, :]).astype(jnp.int32),
                     axis=1)
    last_e = jnp.max(jnp.where(nt > 0, jnp.arange(E, dtype=jnp.int32), -1))
    te = jnp.minimum(te_raw, last_e)
    active = (t_idx < cum[E - 1]).astype(jnp.int32)
    k_t = jnp.clip(t_idx - cumprev[te], 0, TPE - 1)
    xbi_raw = te * TPE + k_t
    xbi_last = last_e * TPE + jnp.maximum(nt[last_e] - 1, 0)
    xbi = jnp.where(active == 1, xbi_raw, xbi_last)

    ys = pl.pallas_call(
        _ffn_body,
        grid_spec=pltpu.PrefetchScalarGridSpec(
            num_scalar_prefetch=3,
            grid=(T_MAX,),
            in_specs=[
                pl.BlockSpec((TILE, D), lambda i, te, act, xbi: (xbi[i], 0)),
                pl.BlockSpec((1, D, 2 * D),
                             lambda i, te, act, xbi: (te[i], 0, 0)),
                pl.BlockSpec((1, 1, 2 * D),
                             lambda i, te, act, xbi: (te[i], 0, 0)),
                pl.BlockSpec((1, 2 * D, D),
                             lambda i, te, act, xbi: (te[i], 0, 0)),
                pl.BlockSpec((1, 1, D),
                             lambda i, te, act, xbi: (te[i], 0, 0)),
            ],
            out_specs=pl.BlockSpec((TILE, D),
                                   lambda i, te, act, xbi: (xbi[i], 0)),
            scratch_shapes=[pltpu.VMEM((D, 2 * D), jnp.bfloat16),
                            pltpu.VMEM((2 * D, D), jnp.bfloat16)],
        ),
        out_shape=jax.ShapeDtypeStruct((P_ALL, D), jnp.float32),
    )(te, active, xbi, xs, W1, b1.reshape(E, 1, 2 * D), W2,
      b2.reshape(E, 1, D))

    pa = jnp.concatenate([pos[:, 0], pos[:, 1]])
    g = sc_gather(ys, pa)

    out_flat = pl.pallas_call(
        _comb_body,
        grid=(N_TOK // BLK,),
        in_specs=[
            pl.BlockSpec((BLK, D), lambda i: (i, 0)),
            pl.BlockSpec((BLK, D), lambda i: (i + N_TOK // BLK, 0)),
            pl.BlockSpec((BLK, K), lambda i: (i, 0)),
        ],
        out_specs=pl.BlockSpec((BLK, D), lambda i: (i, 0)),
        out_shape=jax.ShapeDtypeStruct((N_TOK, D), jnp.float32),
    )(g, g, tw)

    return out_flat.reshape(B, T, D), loss.reshape(())
